# baseline (speedup 1.0000x reference)
import functools

import jax
import jax.numpy as jnp
from jax import lax
from jax.experimental import pallas as pl
from jax.experimental.pallas import tpu as pltpu
from jax.experimental.pallas import tpu_sc as plsc

N = 10000
E = 320000
D = 128
H = 128
C = 64
K = 4

NC = 2
NS = 16
NW = NC * NS
CH = 128
N_PAD = 10112
ROWS_PER_TILE = N_PAD // NS
E_W = 10240
NCH = E_W // CH
NSLAB = 2
SCH = NCH // NSLAB
E_PAD = NW * E_W

_mesh = plsc.VectorSubcoreMesh(core_axis_name="c", subcore_axis_name="s")


def _zero_my_slice(rbuf, acc, s):
    def zfill(i, _):
        rbuf[i // 8, pl.ds((i % 8) * 16, 16)] = jnp.zeros((16,), jnp.float32)
        return 0

    lax.fori_loop(0, CH * 8, zfill, 0)
    base = s * ROWS_PER_TILE
    for q in range(ROWS_PER_TILE // CH):
        pltpu.sync_copy(rbuf, acc.at[pl.ds(base + q * CH, CH)])
    pltpu.sync_copy(rbuf.at[pl.ds(0, ROWS_PER_TILE % CH)],
                    acc.at[pl.ds(base + (ROWS_PER_TILE // CH) * CH,
                                 ROWS_PER_TILE % CH)])


@functools.partial(
    pl.kernel,
    out_type=jax.ShapeDtypeStruct((NC, N_PAD, H), jnp.float32),
    mesh=_mesh,
    scratch_types=[
        pltpu.VMEM((NCH, CH), jnp.int32),
        pltpu.VMEM((CH, H), jnp.float32),
        pltpu.VMEM_SHARED((N_PAD, H), jnp.float32),
    ],
)
def _deg_call(cols3_hbm, out_hbm, colv, rbuf, acc):
    c = lax.axis_index("c")
    s = lax.axis_index("s")
    w = s * NC + c
    base = s * ROWS_PER_TILE

    _zero_my_slice(rbuf, acc, s)

    def ofill(i, _):
        rbuf[i // 8, pl.ds((i % 8) * 16, 16)] = jnp.full((16,), 1.0, jnp.float32)
        return 0

    lax.fori_loop(0, CH * 8, ofill, 0)
    pltpu.sync_copy(cols3_hbm.at[w], colv)
    plsc.subcore_barrier()

    def chunk(j, _):
        pltpu.sync_copy(rbuf, acc.at[colv.at[j]], add=True)
        return 0

    lax.fori_loop(0, NCH, chunk, 0)
    plsc.subcore_barrier()
    pltpu.sync_copy(
        acc.at[pl.ds(base, ROWS_PER_TILE)],
        out_hbm.at[c, pl.ds(base, ROWS_PER_TILE)],
    )


@functools.partial(
    pl.kernel,
    out_type=jax.ShapeDtypeStruct((NC, N_PAD, H), jnp.float32),
    mesh=_mesh,
    scratch_types=[
        pltpu.VMEM((NCH, CH), jnp.int32),
        pltpu.VMEM((NCH, CH), jnp.int32),
        pltpu.VMEM((CH, H), jnp.float32),
        pltpu.VMEM_SHARED((N_PAD, H), jnp.float32),
    ],
)
def _spmm_call(hs_hbm, rows3_hbm, cols3_hbm, out_hbm, rowv, colv, rbuf, acc):
    c = lax.axis_index("c")
    s = lax.axis_index("s")
    w = s * NC + c
    base = s * ROWS_PER_TILE

    _zero_my_slice(rbuf, acc, s)
    pltpu.sync_copy(rows3_hbm.at[w], rowv)
    pltpu.sync_copy(cols3_hbm.at[w], colv)
    plsc.subcore_barrier()

    def chunk(j, _):
        pltpu.sync_copy(hs_hbm.at[rowv.at[j]], rbuf)
        pltpu.sync_copy(rbuf, acc.at[colv.at[j]], add=True)
        return 0

    lax.fori_loop(0, NCH, chunk, 0)
    plsc.subcore_barrier()
    pltpu.sync_copy(
        acc.at[pl.ds(base, ROWS_PER_TILE)],
        out_hbm.at[c, pl.ds(base, ROWS_PER_TILE)],
    )


R = 2528
_GRID = N_PAD // R


def _k0a_body(x_ref, w_ref, b_ref, h_ref):
    h = jnp.dot(x_ref[...], w_ref[...], preferred_element_type=jnp.float32)
    h_ref[...] = jnp.maximum(h + b_ref[...], 0.0)


def _k0b_body(h_ref, degp_ref, hs_ref, nn_ref):
    deg = degp_ref[0, :, 0] + degp_ref[1, :, 0]
    nn = jnp.where(deg > 0.0, lax.rsqrt(deg), 0.0)
    hs_ref[...] = h_ref[...] * nn[:, None]
    nn_ref[...] = nn[:, None]


def _ka_body(h_ref, ctxw_ref, ctxb_ref, wb_ref, z_ref, pre_ref):
    h = h_ref[...]
    logit = jnp.dot(h, ctxw_ref[...], preferred_element_type=jnp.float32)
    logit = logit + ctxb_ref[...]
    m = jnp.max(logit, axis=-1, keepdims=True)
    e = jnp.exp(logit - m)
    z = e / jnp.sum(e, axis=-1, keepdims=True)
    yb = jnp.dot(h, wb_ref[...], preferred_element_type=jnp.float32)
    acc = h
    for k in range(K):
        acc = acc + z[:, k:k + 1] * yb[:, k * H:(k + 1) * H]
    z_ref[...] = z
    pre_ref[...] = acc


def _kb_tail(p_ref, nn_ref, z_ref, pre_ref, wa_ref):
    nn = nn_ref[...]
    hi = (p_ref[0] + p_ref[1]) * nn
    z = z_ref[...]
    ya = jnp.dot(hi, wa_ref[...], preferred_element_type=jnp.float32)
    acc = pre_ref[...]
    for k in range(K):
        acc = acc + z[:, k:k + 1] * ya[:, k * H:(k + 1) * H]
    return jnp.maximum(acc, 0.0), nn


def _k1b_body(p_ref, nn_ref, z_ref, pre_ref, wa_ref, ho_ref, hso_ref):
    ho, nn = _kb_tail(p_ref, nn_ref, z_ref, pre_ref, wa_ref)
    ho_ref[...] = ho
    hso_ref[...] = ho * nn


def _k2b_body(p_ref, nn_ref, z_ref, pre_ref, wa_ref, fc1w_ref, fc1b_ref,
              out_ref):
    ho, _ = _kb_tail(p_ref, nn_ref, z_ref, pre_ref, wa_ref)
    out = jnp.dot(ho, fc1w_ref[...], preferred_element_type=jnp.float32)
    out_ref[...] = out + fc1b_ref[...]


_row_spec = pl.BlockSpec((R, H), lambda i: (i, 0))
_p_spec = pl.BlockSpec((NC, R, H), lambda i: (0, i, 0))
_nn_spec = pl.BlockSpec((R, 1), lambda i: (i, 0))
_z_spec = pl.BlockSpec((R, K), lambda i: (i, 0))
_full = lambda shape: pl.BlockSpec(shape, lambda i: tuple(0 for _ in shape))

_k0a = pl.pallas_call(
    _k0a_body,
    grid=(_GRID,),
    in_specs=[_row_spec, _full((D, H)), _full((1, H))],
    out_specs=_row_spec,
    out_shape=jax.ShapeDtypeStruct((N_PAD, H), jnp.float32),
)

_k0b = pl.pallas_call(
    _k0b_body,
    grid=(_GRID,),
    in_specs=[_row_spec, pl.BlockSpec((NC, R, H), lambda i: (0, i, 0))],
    out_specs=[_row_spec, _nn_spec],
    out_shape=[jax.ShapeDtypeStruct((N_PAD, H), jnp.float32),
               jax.ShapeDtypeStruct((N_PAD, 1), jnp.float32)],
)

_ka = pl.pallas_call(
    _ka_body,
    grid=(_GRID,),
    in_specs=[_row_spec, _full((H, K)), _full((1, K)), _full((H, K * H))],
    out_specs=[_z_spec, _row_spec],
    out_shape=[jax.ShapeDtypeStruct((N_PAD, K), jnp.float32),
               jax.ShapeDtypeStruct((N_PAD, H), jnp.float32)],
)

_k1b = pl.pallas_call(
    _k1b_body,
    grid=(_GRID,),
    in_specs=[_p_spec, _nn_spec, _z_spec, _row_spec, _full((H, K * H))],
    out_specs=[_row_spec, _row_spec],
    out_shape=[jax.ShapeDtypeStruct((N_PAD, H), jnp.float32),
               jax.ShapeDtypeStruct((N_PAD, H), jnp.float32)],
)

_k2b = pl.pallas_call(
    _k2b_body,
    grid=(_GRID,),
    in_specs=[_p_spec, _nn_spec, _z_spec, _row_spec, _full((H, K * H)),
              _full((H, C)), _full((1, C))],
    out_specs=pl.BlockSpec((R, C), lambda i: (i, 0)),
    out_shape=jax.ShapeDtypeStruct((N_PAD, C), jnp.float32),
)


def kernel(x, adj, fc0_W, fc0_b, ctx_W0, ctx_b0, conv_W0, ctx_W1, ctx_b1,
           conv_W1, fc1_W, fc1_b):
    pad = jnp.full((E_PAD - E,), N, dtype=jnp.int32)
    rows3 = jnp.concatenate([adj[0], pad]).reshape(NW, NCH, CH)
    cols3 = jnp.concatenate([adj[1], pad]).reshape(NW, NCH, CH)
    x_p = jnp.concatenate([x, jnp.zeros((N_PAD - N, D), x.dtype)], axis=0)

    degp = _deg_call(cols3)
    h = _k0a(x_p, fc0_W, fc0_b.reshape(1, H))
    hs, nn = _k0b(h, degp)

    for li, (ctx_W, ctx_b, conv_W) in enumerate(
            ((ctx_W0, ctx_b0, conv_W0), (ctx_W1, ctx_b1, conv_W1))):
        wa = jnp.transpose(conv_W[:, :H, :], (1, 0, 2)).reshape(H, K * H)
        wb = jnp.transpose(conv_W[:, H:, :], (1, 0, 2)).reshape(H, K * H)
        p = _spmm_call(hs, rows3, cols3)
        z, pre = _ka(h, ctx_W, ctx_b.reshape(1, K), wb)
        if li == 0:
            h, hs = _k1b(p, nn, z, pre, wa)
        else:
            out = _k2b(p, nn, z, pre, wa, fc1_W, fc1_b.reshape(1, C))
    return out[:N]


# speedup vs baseline: 1.5830x; 1.5830x over previous
import functools

import jax
import jax.numpy as jnp
from jax import lax
from jax.experimental import pallas as pl
from jax.experimental.pallas import tpu as pltpu
from jax.experimental.pallas import tpu_sc as plsc

N = 10000
E = 320000
D = 128
H = 128
C = 64
K = 4

NC = 2
NS = 16
NW = NC * NS
CH = 128
N_PAD = 10112
ROWS_PER_TILE = N_PAD // NS
E_W = 10112
NCH = E_W // CH
E_PAD = NW * E_W

_mesh = plsc.VectorSubcoreMesh(core_axis_name="c", subcore_axis_name="s")


def _zero_my_slice(rbuf, acc, s):
    def zfill(i, _):
        rbuf[i // 8, pl.ds((i % 8) * 16, 16)] = jnp.zeros((16,), jnp.float32)
        return 0

    lax.fori_loop(0, CH * 8, zfill, 0)
    base = s * ROWS_PER_TILE
    for q in range(ROWS_PER_TILE // CH):
        pltpu.sync_copy(rbuf, acc.at[pl.ds(base + q * CH, CH)])
    pltpu.sync_copy(rbuf.at[pl.ds(0, ROWS_PER_TILE % CH)],
                    acc.at[pl.ds(base + (ROWS_PER_TILE // CH) * CH,
                                 ROWS_PER_TILE % CH)])


@functools.partial(
    pl.kernel,
    out_type=jax.ShapeDtypeStruct((NC, N_PAD, H), jnp.float32),
    mesh=_mesh,
    scratch_types=[
        pltpu.VMEM((NCH, CH), jnp.int32),
        pltpu.VMEM((CH, H), jnp.float32),
        pltpu.VMEM_SHARED((N_PAD, H), jnp.float32),
    ],
)
def _deg_call(cols3_hbm, out_hbm, colv, rbuf, acc):
    c = lax.axis_index("c")
    s = lax.axis_index("s")
    w = s * NC + c
    base = s * ROWS_PER_TILE

    _zero_my_slice(rbuf, acc, s)

    def ofill(i, _):
        rbuf[i // 8, pl.ds((i % 8) * 16, 16)] = jnp.full((16,), 1.0, jnp.float32)
        return 0

    lax.fori_loop(0, CH * 8, ofill, 0)
    pltpu.sync_copy(cols3_hbm.at[w], colv)
    plsc.subcore_barrier()

    def chunk(j, _):
        pltpu.sync_copy(rbuf, acc.at[colv.at[j]], add=True)
        return 0

    lax.fori_loop(0, NCH, chunk, 0)
    plsc.subcore_barrier()
    pltpu.sync_copy(
        acc.at[pl.ds(base, ROWS_PER_TILE)],
        out_hbm.at[c, pl.ds(base, ROWS_PER_TILE)],
    )


@functools.partial(
    pl.kernel,
    out_type=jax.ShapeDtypeStruct((NC, N_PAD, H), jnp.float32),
    mesh=_mesh,
    scratch_types=[
        pltpu.VMEM((NCH, CH), jnp.int32),
        pltpu.VMEM((NCH, CH), jnp.int32),
        pltpu.VMEM((CH, H), jnp.float32),
        pltpu.VMEM_SHARED((N_PAD, H), jnp.float32),
    ],
)
def _spmm_call(hs_hbm, rows3_hbm, cols3_hbm, out_hbm, rowv, colv, rbuf, acc):
    c = lax.axis_index("c")
    s = lax.axis_index("s")
    w = s * NC + c
    base = s * ROWS_PER_TILE

    _zero_my_slice(rbuf, acc, s)
    pltpu.sync_copy(rows3_hbm.at[w], rowv)
    pltpu.sync_copy(cols3_hbm.at[w], colv)
    plsc.subcore_barrier()

    def chunk(j, _):
        pltpu.sync_copy(hs_hbm.at[rowv.at[j]], rbuf)
        pltpu.sync_copy(rbuf, acc.at[colv.at[j]], add=True)
        return 0

    lax.fori_loop(0, NCH, chunk, 0)
    plsc.subcore_barrier()
    pltpu.sync_copy(
        acc.at[pl.ds(base, ROWS_PER_TILE)],
        out_hbm.at[c, pl.ds(base, ROWS_PER_TILE)],
    )


R = 2528
_GRID = N_PAD // R


def _k0_body(x_ref, w_ref, b_ref, degp_ref, h_ref, hs_ref, nn_ref):
    deg = degp_ref[0, :, 0] + degp_ref[1, :, 0]
    nn = jnp.where(deg > 0.0, lax.rsqrt(deg), 0.0)
    h = jnp.dot(x_ref[...], w_ref[...], preferred_element_type=jnp.float32)
    h = jnp.maximum(h + b_ref[...], 0.0)
    h_ref[...] = h
    hs_ref[...] = h * nn[:, None]
    nn_ref[...] = nn[:, None]


def _combine(p_ref, h_ref, nn_ref, ctxw_ref, ctxb_ref, wa_ref, wb_ref):
    nn = nn_ref[...]
    h = h_ref[...]
    hi = (p_ref[0] + p_ref[1]) * nn
    logit = jnp.dot(h, ctxw_ref[...], preferred_element_type=jnp.float32)
    logit = logit + ctxb_ref[...]
    m = jnp.max(logit, axis=-1, keepdims=True)
    e = jnp.exp(logit - m)
    z = e / jnp.sum(e, axis=-1, keepdims=True)
    y = jnp.dot(hi, wa_ref[...], preferred_element_type=jnp.float32)
    y = y + jnp.dot(h, wb_ref[...], preferred_element_type=jnp.float32)
    acc = h
    for k in range(K):
        acc = acc + z[:, k:k + 1] * y[:, k * H:(k + 1) * H]
    return jnp.maximum(acc, 0.0), nn


def _k1_body(p_ref, h_ref, nn_ref, ctxw_ref, ctxb_ref, wa_ref, wb_ref,
             ho_ref, hso_ref):
    ho, nn = _combine(p_ref, h_ref, nn_ref, ctxw_ref, ctxb_ref, wa_ref, wb_ref)
    ho_ref[...] = ho
    hso_ref[...] = ho * nn


def _k2_body(p_ref, h_ref, nn_ref, ctxw_ref, ctxb_ref, wa_ref, wb_ref,
             fc1w_ref, fc1b_ref, out_ref):
    ho, _ = _combine(p_ref, h_ref, nn_ref, ctxw_ref, ctxb_ref, wa_ref, wb_ref)
    out = jnp.dot(ho, fc1w_ref[...], preferred_element_type=jnp.float32)
    out_ref[...] = out + fc1b_ref[...]


_row_spec = pl.BlockSpec((R, H), lambda i: (i, 0))
_p_spec = pl.BlockSpec((NC, R, H), lambda i: (0, i, 0))
_nn_spec = pl.BlockSpec((R, 1), lambda i: (i, 0))
_full = lambda shape: pl.BlockSpec(shape, lambda i: tuple(0 for _ in shape))

_k0 = pl.pallas_call(
    _k0_body,
    grid=(_GRID,),
    in_specs=[_row_spec, _full((D, H)), _full((1, H)),
              pl.BlockSpec((NC, R, H), lambda i: (0, i, 0))],
    out_specs=[_row_spec, _row_spec, _nn_spec],
    out_shape=[jax.ShapeDtypeStruct((N_PAD, H), jnp.float32),
               jax.ShapeDtypeStruct((N_PAD, H), jnp.float32),
               jax.ShapeDtypeStruct((N_PAD, 1), jnp.float32)],
)

_k1 = pl.pallas_call(
    _k1_body,
    grid=(_GRID,),
    in_specs=[_p_spec, _row_spec, _nn_spec, _full((H, K)), _full((1, K)),
              _full((H, K * H)), _full((H, K * H))],
    out_specs=[_row_spec, _row_spec],
    out_shape=[jax.ShapeDtypeStruct((N_PAD, H), jnp.float32),
               jax.ShapeDtypeStruct((N_PAD, H), jnp.float32)],
)

_k2 = pl.pallas_call(
    _k2_body,
    grid=(_GRID,),
    in_specs=[_p_spec, _row_spec, _nn_spec, _full((H, K)), _full((1, K)),
              _full((H, K * H)), _full((H, K * H)),
              _full((H, C)), _full((1, C))],
    out_specs=pl.BlockSpec((R, C), lambda i: (i, 0)),
    out_shape=jax.ShapeDtypeStruct((N_PAD, C), jnp.float32),
)


def kernel(x, adj, fc0_W, fc0_b, ctx_W0, ctx_b0, conv_W0, ctx_W1, ctx_b1,
           conv_W1, fc1_W, fc1_b):
    pad = jnp.full((E_PAD - E,), N, dtype=jnp.int32)
    rows3 = jnp.concatenate([adj[0], pad]).reshape(NW, NCH, CH)
    cols3 = jnp.concatenate([adj[1], pad]).reshape(NW, NCH, CH)
    x_p = jnp.concatenate([x, jnp.zeros((N_PAD - N, D), x.dtype)], axis=0)

    degp = _deg_call(cols3)
    h, hs, nn = _k0(x_p, fc0_W, fc0_b.reshape(1, H), degp)

    for li, (ctx_W, ctx_b, conv_W) in enumerate(
            ((ctx_W0, ctx_b0, conv_W0), (ctx_W1, ctx_b1, conv_W1))):
        wa = jnp.transpose(conv_W[:, :H, :], (1, 0, 2)).reshape(H, K * H)
        wb = jnp.transpose(conv_W[:, H:, :], (1, 0, 2)).reshape(H, K * H)
        p = _spmm_call(hs, rows3, cols3)
        if li == 0:
            h, hs = _k1(p, h, nn, ctx_W, ctx_b.reshape(1, K), wa, wb)
        else:
            out = _k2(p, h, nn, ctx_W, ctx_b.reshape(1, K), wa, wb,
                      fc1_W, fc1_b.reshape(1, C))
    return out[:N]
